# baseline (speedup 1.0000x reference)
import functools

import jax
import jax.numpy as jnp
from jax import lax
from jax.experimental import pallas as pl
from jax.experimental.pallas import tpu as pltpu
from jax.experimental.pallas import tpu_sc as plsc

L = 16
NC = 2
NS = 16
NW = NC * NS


def _sc_sims(table, u, v, neg_flat, B, K, D):
    b_per_w = B // NW
    NBLK = 128
    nblocks = b_per_w // NBLK
    nchunk = NBLK * K // 128

    mesh = plsc.VectorSubcoreMesh(core_axis_name="c", subcore_axis_name="s")

    @functools.partial(
        pl.kernel,
        mesh=mesh,
        compiler_params=pltpu.CompilerParams(
            needs_layout_passes=False, use_tc_tiling_on_sc=False),
        out_type=jax.ShapeDtypeStruct(((K + 1) * B,), jnp.float32),
        scratch_types=[
            pltpu.VMEM((NBLK,), jnp.int32),
            pltpu.VMEM((NBLK,), jnp.int32),
            pltpu.VMEM((NBLK * K,), jnp.int32),
            pltpu.VMEM((NBLK, D), jnp.float32),
            pltpu.VMEM((NBLK, D), jnp.float32),
            pltpu.VMEM((NBLK * K, D), jnp.float32),
            pltpu.VMEM(((K + 1) * b_per_w,), jnp.float32),
            pltpu.SemaphoreType.DMA,
        ],
    )
    def sc_kernel(table_h, u_h, v_h, neg_h, sims_h,
                  u_idx, v_idx, neg_idx, u_rows, v_rows, neg_rows, sims_v,
                  sem):
        wid = lax.axis_index("s") * NC + lax.axis_index("c")
        base = wid * b_per_w
        iota = lax.iota(jnp.int32, L)
        for blk in range(nblocks):
            boff = blk * NBLK
            pltpu.sync_copy(u_h.at[pl.ds(base + boff, NBLK)], u_idx)
            pltpu.sync_copy(v_h.at[pl.ds(base + boff, NBLK)], v_idx)
            pltpu.sync_copy(
                neg_h.at[pl.ds((base + boff) * K, NBLK * K)], neg_idx)
            cps = [pltpu.async_copy(table_h.at[u_idx], u_rows, sem),
                   pltpu.async_copy(table_h.at[v_idx], v_rows, sem)]
            for j in range(nchunk):
                cps.append(pltpu.async_copy(
                    table_h.at[neg_idx.at[pl.ds(j * 128, 128)]],
                    neg_rows.at[pl.ds(j * 128, 128), :], sem))
            for c in cps:
                c.wait()
            for g in range(NBLK // L):
                r = g * L + iota
                rk = r * K
                def body(d, accs, r=r, rk=rk):
                    dv = jnp.full((L,), 0, jnp.int32) + d
                    uu = plsc.load_gather(u_rows, [r, dv])
                    vv = plsc.load_gather(v_rows, [r, dv])
                    out = [accs[0] + uu * vv]
                    for k in range(K):
                        nn = plsc.load_gather(neg_rows, [rk + k, dv])
                        out.append(accs[k + 1] + vv * nn)
                    return tuple(out)
                accs = lax.fori_loop(
                    0, D, body,
                    tuple(jnp.zeros((L,), jnp.float32) for _ in range(K + 1)))
                sims_v[pl.ds(boff + g * L, L)] = -accs[0]
                for k in range(K):
                    sims_v[pl.ds((1 + k) * b_per_w + boff + g * L, L)] = accs[k + 1]
        for j in range(K + 1):
            pltpu.sync_copy(sims_v.at[pl.ds(j * b_per_w, b_per_w)],
                            sims_h.at[pl.ds(j * B + base, b_per_w)])

    return sc_kernel(table, u, v, neg_flat)


def _tc_logsigmoid_sum(sims, B, K):
    BT = 2048

    def body(s_ref, o_ref):
        x = s_ref[...]
        sp = jnp.maximum(x, 0.0) + jnp.log1p(jnp.exp(-jnp.abs(x)))
        o_ref[...] = jnp.sum(sp, axis=0, keepdims=True)

    return pl.pallas_call(
        body,
        grid=(B // BT,),
        in_specs=[pl.BlockSpec((K + 1, BT), lambda i: (0, i))],
        out_specs=pl.BlockSpec((1, BT), lambda i: (0, i)),
        out_shape=jax.ShapeDtypeStruct((1, B), jnp.float32),
    )(sims)


def kernel(u, v, neg, table):
    B = u.shape[0]
    K = neg.shape[1]
    D = table.shape[1]
    u32 = u.astype(jnp.int32)
    v32 = v.astype(jnp.int32)
    neg_flat = neg.astype(jnp.int32).reshape(-1)
    sims = _sc_sims(table, u32, v32, neg_flat, B, K, D).reshape(K + 1, B)
    p = _tc_logsigmoid_sum(sims, B, K)
    return p.reshape(B)


# speedup vs baseline: 1.2945x; 1.2945x over previous
import functools

import jax
import jax.numpy as jnp
from jax import lax
from jax.experimental import pallas as pl
from jax.experimental.pallas import tpu as pltpu
from jax.experimental.pallas import tpu_sc as plsc

L = 16
NC = 2
NS = 16
NW = NC * NS


def _sc_sims(table, u, v, neg_flat, B, K, D):
    b_per_w = B // NW
    NBLK = 128
    nblocks = b_per_w // NBLK
    nchunk = NBLK * K // 128

    mesh = plsc.VectorSubcoreMesh(core_axis_name="c", subcore_axis_name="s")

    @functools.partial(
        pl.kernel,
        mesh=mesh,
        compiler_params=pltpu.CompilerParams(
            needs_layout_passes=False, use_tc_tiling_on_sc=False),
        out_type=jax.ShapeDtypeStruct(((K + 1) * B,), jnp.float32),
        scratch_types=[
            pltpu.VMEM((NBLK,), jnp.int32),
            pltpu.VMEM((NBLK,), jnp.int32),
            pltpu.VMEM((NBLK * K,), jnp.int32),
            pltpu.VMEM((NBLK, D), jnp.float32),
            pltpu.VMEM((NBLK, D), jnp.float32),
            pltpu.VMEM((NBLK * K, D), jnp.float32),
            pltpu.VMEM(((K + 1) * b_per_w,), jnp.float32),
            pltpu.SemaphoreType.DMA,
        ],
    )
    def sc_kernel(table_h, u_h, v_h, neg_h, sims_h,
                  u_idx, v_idx, neg_idx, u_rows, v_rows, neg_rows, sims_v,
                  sem):
        wid = lax.axis_index("s") * NC + lax.axis_index("c")
        base = wid * b_per_w
        iota = lax.iota(jnp.int32, L)
        for blk in range(nblocks):
            boff = blk * NBLK
            pltpu.sync_copy(u_h.at[pl.ds(base + boff, NBLK)], u_idx)
            pltpu.sync_copy(v_h.at[pl.ds(base + boff, NBLK)], v_idx)
            pltpu.sync_copy(
                neg_h.at[pl.ds((base + boff) * K, NBLK * K)], neg_idx)
            cps = [pltpu.async_copy(table_h.at[u_idx], u_rows, sem),
                   pltpu.async_copy(table_h.at[v_idx], v_rows, sem)]
            for j in range(nchunk):
                cps.append(pltpu.async_copy(
                    table_h.at[neg_idx.at[pl.ds(j * 128, 128)]],
                    neg_rows.at[pl.ds(j * 128, 128), :], sem))
            for c in cps:
                c.wait()
            nch = D // L
            for g in range(NBLK // L):
                def body(lb, accs, g=g):
                    row = g * L + lb
                    vvec = [v_rows[row, pl.ds(c * L, L)] for c in range(nch)]
                    uvec = [u_rows[row, pl.ds(c * L, L)] for c in range(nch)]
                    s = uvec[0] * vvec[0]
                    for c in range(1, nch):
                        s = s + uvec[c] * vvec[c]
                    lane = iota == lb
                    out = [jnp.where(lane, jnp.sum(s), accs[0])]
                    for k in range(K):
                        nr = row * K + k
                        s = vvec[0] * neg_rows[nr, pl.ds(0, L)]
                        for c in range(1, nch):
                            s = s + vvec[c] * neg_rows[nr, pl.ds(c * L, L)]
                        out.append(jnp.where(lane, jnp.sum(s), accs[k + 1]))
                    return tuple(out)
                accs = lax.fori_loop(
                    0, L, body,
                    tuple(jnp.zeros((L,), jnp.float32) for _ in range(K + 1)))
                sims_v[pl.ds(boff + g * L, L)] = -accs[0]
                for k in range(K):
                    sims_v[pl.ds((1 + k) * b_per_w + boff + g * L, L)] = accs[k + 1]
        for j in range(K + 1):
            pltpu.sync_copy(sims_v.at[pl.ds(j * b_per_w, b_per_w)],
                            sims_h.at[pl.ds(j * B + base, b_per_w)])

    return sc_kernel(table, u, v, neg_flat)


def _tc_logsigmoid_sum(sims, B, K):
    BT = 2048

    def body(s_ref, o_ref):
        x = s_ref[...]
        sp = jnp.maximum(x, 0.0) + jnp.log1p(jnp.exp(-jnp.abs(x)))
        o_ref[...] = jnp.sum(sp, axis=0, keepdims=True)

    return pl.pallas_call(
        body,
        grid=(B // BT,),
        in_specs=[pl.BlockSpec((K + 1, BT), lambda i: (0, i))],
        out_specs=pl.BlockSpec((1, BT), lambda i: (0, i)),
        out_shape=jax.ShapeDtypeStruct((1, B), jnp.float32),
    )(sims)


def kernel(u, v, neg, table):
    B = u.shape[0]
    K = neg.shape[1]
    D = table.shape[1]
    u32 = u.astype(jnp.int32)
    v32 = v.astype(jnp.int32)
    neg_flat = neg.astype(jnp.int32).reshape(-1)
    sims = _sc_sims(table, u32, v32, neg_flat, B, K, D).reshape(K + 1, B)
    p = _tc_logsigmoid_sum(sims, B, K)
    return p.reshape(B)
